# baseline (speedup 1.0000x reference)
import jax
import jax.numpy as jnp
from jax.experimental import pallas as pl
from jax.experimental.pallas import tpu as pltpu

_B, _N, _C = 16, 5000, 200
_BN = 1000
_NBLK = _N // _BN
_GRID = _B * _NBLK


def _body(vd_ref, off_ref, logits_ref, center_ref, width_ref,
          scores_ref, labels_ref, t1_ref, t2_ref, mask_ref):
    g = pl.program_id(0)
    b = g // _NBLK

    x = logits_ref[0]
    m = jnp.max(x, axis=1, keepdims=True)
    ids = jax.lax.broadcasted_iota(jnp.int32, x.shape, 1)
    lbl = jnp.min(jnp.where(x == m, ids, _C), axis=1, keepdims=True)
    scores_ref[0] = jax.nn.sigmoid(m)
    labels_ref[0] = lbl

    off = off_ref[b]
    vd = vd_ref[b]
    c = center_ref[0]
    half_w = 0.5 * jnp.exp(width_ref[0])
    t1 = jnp.clip(c - half_w + off, 0.0, vd)
    t2 = jnp.clip(c + half_w + off, 0.0, vd)
    t1_ref[0] = t1
    t2_ref[0] = t2
    mask_ref[0] = ((t2 - t1) > 0.05).astype(jnp.int32)


@jax.jit
def kernel(pred_logits, pred_segments, video_durations, feature_durations,
           strides, offsets):
    del feature_durations, strides
    logits3 = pred_logits.reshape(_GRID, _BN, _C)
    center3 = pred_segments[:, :, 0].reshape(_GRID, _BN, 1)
    width3 = pred_segments[:, :, 1].reshape(_GRID, _BN, 1)

    col_spec = pl.BlockSpec((1, _BN, 1), lambda g: (g, 0, 0))
    smem_spec = pl.BlockSpec(memory_space=pltpu.SMEM)
    out_sds = jax.ShapeDtypeStruct((_GRID, _BN, 1), jnp.float32)
    out_ids = jax.ShapeDtypeStruct((_GRID, _BN, 1), jnp.int32)

    scores, labels, t1, t2, mask = pl.pallas_call(
        _body,
        grid=(_GRID,),
        in_specs=[
            smem_spec,
            smem_spec,
            pl.BlockSpec((1, _BN, _C), lambda g: (g, 0, 0)),
            col_spec,
            col_spec,
        ],
        out_specs=[col_spec] * 5,
        out_shape=[out_sds, out_ids, out_sds, out_sds, out_ids],
    )(video_durations, offsets, logits3, center3, width3)

    scores = scores.reshape(_B, _N)
    labels = labels.reshape(_B, _N)
    segments = jnp.stack([t1.reshape(_B, _N), t2.reshape(_B, _N)], axis=-1)
    valid_mask = mask.reshape(_B, _N).astype(bool)
    return scores, labels, segments, valid_mask


# speedup vs baseline: 1.0614x; 1.0614x over previous
import jax
import jax.numpy as jnp
from jax.experimental import pallas as pl
from jax.experimental.pallas import tpu as pltpu

_B, _N, _C = 16, 5000, 200
_BN = 1000
_NBLK = _N // _BN
_GRID = _B * _NBLK


def _body(vd_ref, off_ref, logits_ref, seg_ref,
          scores_ref, labels_ref, segs_ref, mask_ref):
    g = pl.program_id(0)
    b = g // _NBLK

    x = logits_ref[0]
    m = jnp.max(x, axis=1, keepdims=True)
    ids = jax.lax.broadcasted_iota(jnp.int32, x.shape, 1)
    lbl = jnp.min(jnp.where(x == m, ids, _C), axis=1, keepdims=True)
    scores_ref[0] = jax.nn.sigmoid(m)
    labels_ref[0] = lbl

    off = off_ref[b]
    vd = vd_ref[b]
    c = seg_ref[0, :, 0:1]
    half_w = 0.5 * jnp.exp(seg_ref[0, :, 1:2])
    t1 = jnp.clip(c - half_w + off, 0.0, vd)
    t2 = jnp.clip(c + half_w + off, 0.0, vd)
    segs_ref[0, :, 0:1] = t1
    segs_ref[0, :, 1:2] = t2
    mask_ref[0] = ((t2 - t1) > 0.05).astype(jnp.int32)


@jax.jit
def kernel(pred_logits, pred_segments, video_durations, feature_durations,
           strides, offsets):
    del feature_durations, strides
    logits3 = pred_logits.reshape(_GRID, _BN, _C)
    seg3 = pred_segments.reshape(_GRID, _BN, 2)

    col_spec = pl.BlockSpec((1, _BN, 1), lambda g: (g, 0, 0))
    seg_spec = pl.BlockSpec((1, _BN, 2), lambda g: (g, 0, 0))
    smem_spec = pl.BlockSpec(memory_space=pltpu.SMEM)

    scores, labels, segs, mask = pl.pallas_call(
        _body,
        grid=(_GRID,),
        in_specs=[
            smem_spec,
            smem_spec,
            pl.BlockSpec((1, _BN, _C), lambda g: (g, 0, 0)),
            seg_spec,
        ],
        out_specs=[col_spec, col_spec, seg_spec, col_spec],
        out_shape=[
            jax.ShapeDtypeStruct((_GRID, _BN, 1), jnp.float32),
            jax.ShapeDtypeStruct((_GRID, _BN, 1), jnp.int32),
            jax.ShapeDtypeStruct((_GRID, _BN, 2), jnp.float32),
            jax.ShapeDtypeStruct((_GRID, _BN, 1), jnp.int32),
        ],
    )(video_durations, offsets, logits3, seg3)

    scores = scores.reshape(_B, _N)
    labels = labels.reshape(_B, _N)
    segments = segs.reshape(_B, _N, 2)
    valid_mask = mask.reshape(_B, _N).astype(bool)
    return scores, labels, segments, valid_mask


# speedup vs baseline: 3.2481x; 3.0601x over previous
import jax
import jax.numpy as jnp
from jax.experimental import pallas as pl
from jax.experimental.pallas import tpu as pltpu

_B, _N, _C = 16, 5000, 200
_BN = 1000
_NBLK = _N // _BN


def _body(vd_ref, off_ref, logits_ref, seg_ref,
          scores_ref, labels_ref, segs_ref, mask_ref):
    b = pl.program_id(0)
    row = pl.ds(b, 1)

    ms, lbls = [], []
    for nb in range(_NBLK):
        xt = logits_ref[0, pl.ds(nb * _BN, _BN), :].T
        m = jnp.max(xt, axis=0, keepdims=True)
        ids = jax.lax.broadcasted_iota(
            jnp.int32, xt.shape, 0).astype(jnp.float32)
        lbls.append(jnp.min(jnp.where(xt == m, ids, float(_C)),
                            axis=0, keepdims=True))
        ms.append(m)
    scores_ref[row, :] = jax.nn.sigmoid(jnp.concatenate(ms, axis=1))
    labels_ref[row, :] = jnp.concatenate(lbls, axis=1).astype(jnp.int32)

    off = off_ref[b]
    vd = vd_ref[b]
    segt = seg_ref[0].T
    c = segt[0:1, :]
    half_w = 0.5 * jnp.exp(segt[1:2, :])
    t1 = jnp.clip(c - half_w + off, 0.0, vd)
    t2 = jnp.clip(c + half_w + off, 0.0, vd)
    segs_ref[row, :, :] = jnp.concatenate([t1, t2], axis=0).T[None]
    mask_ref[row, :] = (t2 - t1) > 0.05


@jax.jit
def kernel(pred_logits, pred_segments, video_durations, feature_durations,
           strides, offsets):
    del feature_durations, strides

    smem_spec = pl.BlockSpec(memory_space=pltpu.SMEM)

    scores, labels, segments, valid_mask = pl.pallas_call(
        _body,
        grid=(_B,),
        in_specs=[
            smem_spec,
            smem_spec,
            pl.BlockSpec((1, _N, _C), lambda b: (b, 0, 0)),
            pl.BlockSpec((1, _N, 2), lambda b: (b, 0, 0)),
        ],
        out_specs=[
            pl.BlockSpec((_B, _N), lambda b: (0, 0)),
            pl.BlockSpec((_B, _N), lambda b: (0, 0)),
            pl.BlockSpec((_B, _N, 2), lambda b: (0, 0, 0)),
            pl.BlockSpec((_B, _N), lambda b: (0, 0)),
        ],
        out_shape=[
            jax.ShapeDtypeStruct((_B, _N), jnp.float32),
            jax.ShapeDtypeStruct((_B, _N), jnp.int32),
            jax.ShapeDtypeStruct((_B, _N, 2), jnp.float32),
            jax.ShapeDtypeStruct((_B, _N), jnp.bool_),
        ],
    )(video_durations, offsets, pred_logits, pred_segments)

    return scores, labels, segments, valid_mask


# speedup vs baseline: 14.3018x; 4.4032x over previous
import jax
import jax.numpy as jnp
from jax.experimental import pallas as pl
from jax.experimental.pallas import tpu as pltpu

_B, _N, _C = 16, 5000, 200


def _body(vd_ref, off_ref, logits_ref, seg_ref,
          scores_ref, labels_ref, segs_ref, mask_ref):
    b = pl.program_id(0)
    row = pl.ds(b, 1)

    xt = logits_ref[0]
    m = jnp.max(xt, axis=0, keepdims=True)
    scores_ref[row, :] = jax.nn.sigmoid(m)
    onehot = (xt == m).astype(jnp.float32)
    ids = jax.lax.broadcasted_iota(
        jnp.int32, (1, _C), 1).astype(jnp.float32)
    w = jnp.concatenate([jnp.ones((1, _C), jnp.float32), ids, ids * ids],
                        axis=0)
    r = jax.lax.dot_general(w, onehot, (((1,), (0,)), ((), ())),
                            precision=jax.lax.Precision.HIGHEST,
                            preferred_element_type=jnp.float32)
    cnt, s, q = r[0:1], r[1:2], r[2:3]
    tie = (s - jnp.sqrt(jnp.maximum(2.0 * q - s * s, 0.0))) * 0.5
    lbl = jnp.where(cnt > 1.5, tie, s)
    labels_ref[row, :] = lbl.astype(jnp.int32)

    off = off_ref[b]
    vd = vd_ref[b]
    st = seg_ref[0]
    c = st[0:1, :]
    half_w = 0.5 * jnp.exp(st[1:2, :])
    t1 = jnp.clip(c - half_w + off, 0.0, vd)
    t2 = jnp.clip(c + half_w + off, 0.0, vd)
    segs_ref[row, 0:1, :] = t1[None]
    segs_ref[row, 1:2, :] = t2[None]
    mask_ref[row, :] = (t2 - t1) > 0.05


@jax.jit
def kernel(pred_logits, pred_segments, video_durations, feature_durations,
           strides, offsets):
    del feature_durations, strides
    lt = jnp.transpose(pred_logits, (0, 2, 1))
    st = jnp.transpose(pred_segments, (0, 2, 1))

    smem_spec = pl.BlockSpec(memory_space=pltpu.SMEM)

    scores, labels, segs2, valid_mask = pl.pallas_call(
        _body,
        grid=(_B,),
        in_specs=[
            smem_spec,
            smem_spec,
            pl.BlockSpec((1, _C, _N), lambda b: (b, 0, 0)),
            pl.BlockSpec((1, 2, _N), lambda b: (b, 0, 0)),
        ],
        out_specs=[
            pl.BlockSpec((_B, _N), lambda b: (0, 0)),
            pl.BlockSpec((_B, _N), lambda b: (0, 0)),
            pl.BlockSpec((_B, 2, _N), lambda b: (0, 0, 0)),
            pl.BlockSpec((_B, _N), lambda b: (0, 0)),
        ],
        out_shape=[
            jax.ShapeDtypeStruct((_B, _N), jnp.float32),
            jax.ShapeDtypeStruct((_B, _N), jnp.int32),
            jax.ShapeDtypeStruct((_B, 2, _N), jnp.float32),
            jax.ShapeDtypeStruct((_B, _N), jnp.bool_),
        ],
    )(video_durations, offsets, lt, st)

    segments = jnp.transpose(segs2, (0, 2, 1))
    return scores, labels, segments, valid_mask


# speedup vs baseline: 19.9284x; 1.3934x over previous
import jax
import jax.numpy as jnp
from jax.experimental import pallas as pl
from jax.experimental.pallas import tpu as pltpu

_B, _N, _C = 16, 5000, 200


def _body(vd_ref, off_ref, w_ref, logits_ref, seg_ref,
          scores_ref, labels_ref, segs_ref, mask_ref):
    b = pl.program_id(0)
    row = pl.ds(b, 1)

    xt = logits_ref[0]
    m = jnp.max(xt, axis=0, keepdims=True)
    scores_ref[row, :] = jax.nn.sigmoid(m)
    onehot = (xt == m).astype(jnp.bfloat16)
    r = jax.lax.dot_general(w_ref[...], onehot, (((1,), (0,)), ((), ())),
                            preferred_element_type=jnp.float32)
    cnt, s = r[0:1], r[1:2]
    q = r[2:3] + r[3:4]
    tie = (s - jnp.sqrt(jnp.maximum(2.0 * q - s * s, 0.0))) * 0.5
    lbl = jnp.where(cnt > 1.5, tie, s)
    labels_ref[row, :] = lbl.astype(jnp.int32)

    off = off_ref[b]
    vd = vd_ref[b]
    st = seg_ref[0]
    c = st[0:1, :]
    half_w = 0.5 * jnp.exp(st[1:2, :])
    t1 = jnp.clip(c - half_w + off, 0.0, vd)
    t2 = jnp.clip(c + half_w + off, 0.0, vd)
    segs_ref[row, 0:1, :] = t1[None]
    segs_ref[row, 1:2, :] = t2[None]
    mask_ref[row, :] = (t2 - t1) > 0.05


@jax.jit
def kernel(pred_logits, pred_segments, video_durations, feature_durations,
           strides, offsets):
    del feature_durations, strides
    lt = jnp.transpose(pred_logits, (0, 2, 1))
    st = jnp.transpose(pred_segments, (0, 2, 1))

    ids = jnp.arange(_C, dtype=jnp.float32)
    q = ids * ids
    q_hi = q.astype(jnp.bfloat16)
    q_lo = (q - q_hi.astype(jnp.float32)).astype(jnp.bfloat16)
    w = jnp.stack([jnp.ones((_C,), jnp.bfloat16), ids.astype(jnp.bfloat16),
                   q_hi, q_lo])

    smem_spec = pl.BlockSpec(memory_space=pltpu.SMEM)

    scores, labels, segs2, valid_mask = pl.pallas_call(
        _body,
        grid=(_B,),
        in_specs=[
            smem_spec,
            smem_spec,
            pl.BlockSpec((4, _C), lambda b: (0, 0)),
            pl.BlockSpec((1, _C, _N), lambda b: (b, 0, 0)),
            pl.BlockSpec((1, 2, _N), lambda b: (b, 0, 0)),
        ],
        out_specs=[
            pl.BlockSpec((_B, _N), lambda b: (0, 0)),
            pl.BlockSpec((_B, _N), lambda b: (0, 0)),
            pl.BlockSpec((_B, 2, _N), lambda b: (0, 0, 0)),
            pl.BlockSpec((_B, _N), lambda b: (0, 0)),
        ],
        out_shape=[
            jax.ShapeDtypeStruct((_B, _N), jnp.float32),
            jax.ShapeDtypeStruct((_B, _N), jnp.int32),
            jax.ShapeDtypeStruct((_B, 2, _N), jnp.float32),
            jax.ShapeDtypeStruct((_B, _N), jnp.bool_),
        ],
    )(video_durations, offsets, w, lt, st)

    segments = jnp.transpose(segs2, (0, 2, 1))
    return scores, labels, segments, valid_mask


# speedup vs baseline: 20.4291x; 1.0251x over previous
import jax
import jax.numpy as jnp
import numpy as np
from jax.experimental import pallas as pl
from jax.experimental.pallas import tpu as pltpu

_B, _N, _C = 16, 5000, 200

def _make_argmax_weights() -> np.ndarray:
    import ml_dtypes
    bf16 = ml_dtypes.bfloat16
    ids = np.arange(_C, dtype=np.float32)
    q = ids * ids
    q_hi = q.astype(bf16)
    q_lo = (q - q_hi.astype(np.float32)).astype(bf16)
    return np.stack([np.ones((_C,), bf16), ids.astype(bf16), q_hi, q_lo])


_W_NP = _make_argmax_weights()


def _body(vd_ref, off_ref, w_ref, logits_ref, seg_ref,
          scores_ref, labels_ref, segs_ref, mask_ref):
    b = pl.program_id(0)
    row = pl.ds(b, 1)

    xt = logits_ref[0]
    m = jnp.max(xt, axis=0, keepdims=True)
    scores_ref[row, :] = jax.nn.sigmoid(m)
    onehot = (xt == m).astype(jnp.bfloat16)
    r = jax.lax.dot_general(w_ref[...], onehot, (((1,), (0,)), ((), ())),
                            preferred_element_type=jnp.float32)
    cnt, s = r[0:1], r[1:2]
    q = r[2:3] + r[3:4]
    tie = (s - jnp.sqrt(jnp.maximum(2.0 * q - s * s, 0.0))) * 0.5
    lbl = jnp.where(cnt > 1.5, tie, s)
    labels_ref[row, :] = lbl.astype(jnp.int32)

    off = off_ref[b]
    vd = vd_ref[b]
    st = seg_ref[0]
    c = st[0:1, :]
    half_w = 0.5 * jnp.exp(st[1:2, :])
    t1 = jnp.clip(c - half_w + off, 0.0, vd)
    t2 = jnp.clip(c + half_w + off, 0.0, vd)
    segs_ref[row, 0:1, :] = t1[None]
    segs_ref[row, 1:2, :] = t2[None]
    mask_ref[row, :] = (t2 - t1) > 0.05


@jax.jit
def kernel(pred_logits, pred_segments, video_durations, feature_durations,
           strides, offsets):
    del feature_durations, strides
    lt = jnp.transpose(pred_logits, (0, 2, 1))
    st = jnp.transpose(pred_segments, (0, 2, 1))

    w = jnp.asarray(_W_NP)

    smem_spec = pl.BlockSpec(memory_space=pltpu.SMEM)

    scores, labels, segs2, valid_mask = pl.pallas_call(
        _body,
        grid=(_B,),
        in_specs=[
            smem_spec,
            smem_spec,
            pl.BlockSpec((4, _C), lambda b: (0, 0)),
            pl.BlockSpec((1, _C, _N), lambda b: (b, 0, 0)),
            pl.BlockSpec((1, 2, _N), lambda b: (b, 0, 0)),
        ],
        out_specs=[
            pl.BlockSpec((_B, _N), lambda b: (0, 0)),
            pl.BlockSpec((_B, _N), lambda b: (0, 0)),
            pl.BlockSpec((_B, 2, _N), lambda b: (0, 0, 0)),
            pl.BlockSpec((_B, _N), lambda b: (0, 0)),
        ],
        out_shape=[
            jax.ShapeDtypeStruct((_B, _N), jnp.float32),
            jax.ShapeDtypeStruct((_B, _N), jnp.int32),
            jax.ShapeDtypeStruct((_B, 2, _N), jnp.float32),
            jax.ShapeDtypeStruct((_B, _N), jnp.bool_),
        ],
    )(video_durations, offsets, w, lt, st)

    segments = jnp.transpose(segs2, (0, 2, 1))
    return scores, labels, segments, valid_mask


# speedup vs baseline: 22.3169x; 1.0924x over previous
import jax
import jax.numpy as jnp
import numpy as np
from jax.experimental import pallas as pl
from jax.experimental.pallas import tpu as pltpu

_B, _N, _C = 16, 5000, 200
_BPG = 2
_G = _B // _BPG

def _make_argmax_weights() -> np.ndarray:
    import ml_dtypes
    bf16 = ml_dtypes.bfloat16
    ids = np.arange(_C, dtype=np.float32)
    q = ids * ids
    q_hi = q.astype(bf16)
    q_lo = (q - q_hi.astype(np.float32)).astype(bf16)
    return np.stack([np.ones((_C,), bf16), ids.astype(bf16), q_hi, q_lo])


_W_NP = _make_argmax_weights()


def _body(vd_ref, off_ref, w_ref, logits_ref, seg_ref,
          scores_ref, labels_ref, segs_ref, mask_ref):
  g = pl.program_id(0)
  for i in range(_BPG):
    b = g * _BPG + i
    row = pl.ds(b, 1)

    xt = logits_ref[i]
    m = jnp.max(xt, axis=0, keepdims=True)
    scores_ref[row, :] = jax.nn.sigmoid(m)
    onehot = (xt == m).astype(jnp.bfloat16)
    r = jax.lax.dot_general(w_ref[...], onehot, (((1,), (0,)), ((), ())),
                            preferred_element_type=jnp.float32)
    cnt, s = r[0:1], r[1:2]
    q = r[2:3] + r[3:4]
    tie = (s - jnp.sqrt(jnp.maximum(2.0 * q - s * s, 0.0))) * 0.5
    lbl = jnp.where(cnt > 1.5, tie, s)
    labels_ref[row, :] = lbl.astype(jnp.int32)

    off = off_ref[b]
    vd = vd_ref[b]
    st = seg_ref[i]
    c = st[0:1, :]
    half_w = 0.5 * jnp.exp(st[1:2, :])
    t1 = jnp.clip(c - half_w + off, 0.0, vd)
    t2 = jnp.clip(c + half_w + off, 0.0, vd)
    segs_ref[row, 0:1, :] = t1[None]
    segs_ref[row, 1:2, :] = t2[None]
    mask_ref[row, :] = (t2 - t1) > 0.05


@jax.jit
def kernel(pred_logits, pred_segments, video_durations, feature_durations,
           strides, offsets):
    del feature_durations, strides
    lt = jnp.transpose(pred_logits, (0, 2, 1))
    st = jnp.transpose(pred_segments, (0, 2, 1))

    w = jnp.asarray(_W_NP)

    smem_spec = pl.BlockSpec(memory_space=pltpu.SMEM)

    scores, labels, segs2, valid_mask = pl.pallas_call(
        _body,
        grid=(_G,),
        in_specs=[
            smem_spec,
            smem_spec,
            pl.BlockSpec((4, _C), lambda g: (0, 0)),
            pl.BlockSpec((_BPG, _C, _N), lambda g: (g, 0, 0)),
            pl.BlockSpec((_BPG, 2, _N), lambda g: (g, 0, 0)),
        ],
        out_specs=[
            pl.BlockSpec((_B, _N), lambda g: (0, 0)),
            pl.BlockSpec((_B, _N), lambda g: (0, 0)),
            pl.BlockSpec((_B, 2, _N), lambda g: (0, 0, 0)),
            pl.BlockSpec((_B, _N), lambda g: (0, 0)),
        ],
        out_shape=[
            jax.ShapeDtypeStruct((_B, _N), jnp.float32),
            jax.ShapeDtypeStruct((_B, _N), jnp.int32),
            jax.ShapeDtypeStruct((_B, 2, _N), jnp.float32),
            jax.ShapeDtypeStruct((_B, _N), jnp.bool_),
        ],
    )(video_durations, offsets, w, lt, st)

    segments = jnp.transpose(segs2, (0, 2, 1))
    return scores, labels, segments, valid_mask
